# baseline (speedup 1.0000x reference)
import functools

import jax
import jax.numpy as jnp
from jax import lax
from jax.experimental import pallas as pl
from jax.experimental.pallas import tpu as pltpu
from jax.experimental.pallas import tpu_sc as plsc

N = 10000
E = 320000
D = 128
NRBF = 32
G = 256
NZ = 90
CUTOFF = 6.0

NC = 2
NS = 16
NW = NC * NS
EP = 327680
EW = EP // NW
K = 80
NCHUNK = EW // K
KE = 128
NCHUNKE = EW // KE
NP = 10240
NROWS = NP // NS


def _rsqrt16(x):
    i = lax.bitcast_convert_type(x, jnp.int32)
    i = jnp.int32(0x5F3759DF) - lax.shift_right_arithmetic(i, 1)
    y = lax.bitcast_convert_type(i, jnp.float32)
    for _ in range(3):
        y = y * (1.5 - 0.5 * x * y * y)
    return y


def _geom_body(ew, src_hbm, dst_hbm, z_hbm, px_hbm, py_hbm, pz_hbm,
               dist_hbm, zsrc_hbm, ux_hbm, uy_hbm, uz_hbm,
               px_v, py_v, pz_v, z_v, src_v, dst_v, dist_v, zs_v,
               ux_v, uy_v, uz_v):
    EW = ew
    cid = lax.axis_index("c")
    sid = lax.axis_index("s")
    wid = sid * NC + cid
    base = wid * EW

    pltpu.sync_copy(px_hbm, px_v)
    pltpu.sync_copy(py_hbm, py_v)
    pltpu.sync_copy(pz_hbm, pz_v)
    pltpu.sync_copy(z_hbm, z_v)
    pltpu.sync_copy(src_hbm.at[pl.ds(base, EW)], src_v)
    pltpu.sync_copy(dst_hbm.at[pl.ds(base, EW)], dst_v)

    def body(g, carry):
        off = g * 16
        s16 = src_v[pl.ds(off, 16)]
        d16 = dst_v[pl.ds(off, 16)]
        dx = plsc.load_gather(px_v, [s16]) - plsc.load_gather(px_v, [d16])
        dy = plsc.load_gather(py_v, [s16]) - plsc.load_gather(py_v, [d16])
        dz = plsc.load_gather(pz_v, [s16]) - plsc.load_gather(pz_v, [d16])
        zx = plsc.load_gather(z_v, [s16])
        sq = dx * dx + dy * dy + dz * dz + 1e-8
        r = _rsqrt16(sq)
        dist_v[pl.ds(off, 16)] = sq * r
        zs_v[pl.ds(off, 16)] = zx
        ux_v[pl.ds(off, 16)] = dx * r
        uy_v[pl.ds(off, 16)] = dy * r
        uz_v[pl.ds(off, 16)] = dz * r
        return carry

    lax.fori_loop(0, EW // 16, body, 0)
    pltpu.sync_copy(dist_v, dist_hbm.at[pl.ds(base, EW)])
    pltpu.sync_copy(zs_v, zsrc_hbm.at[pl.ds(base, EW)])
    pltpu.sync_copy(ux_v, ux_hbm.at[pl.ds(base, EW)])
    pltpu.sync_copy(uy_v, uy_hbm.at[pl.ds(base, EW)])
    pltpu.sync_copy(uz_v, uz_hbm.at[pl.ds(base, EW)])


def _make_geom(el):
    ew = el // NW
    return pl.kernel(
        functools.partial(_geom_body, ew),
        out_type=(jax.ShapeDtypeStruct((el,), jnp.float32),
                  jax.ShapeDtypeStruct((el,), jnp.int32),
                  jax.ShapeDtypeStruct((el,), jnp.float32),
                  jax.ShapeDtypeStruct((el,), jnp.float32),
                  jax.ShapeDtypeStruct((el,), jnp.float32)),
        mesh=plsc.VectorSubcoreMesh(core_axis_name="c", subcore_axis_name="s"),
        compiler_params=pltpu.CompilerParams(needs_layout_passes=False),
        scratch_types=[
            pltpu.VMEM((N,), jnp.float32),
            pltpu.VMEM((N,), jnp.float32),
            pltpu.VMEM((N,), jnp.float32),
            pltpu.VMEM((N,), jnp.int32),
            pltpu.VMEM((ew,), jnp.int32),
            pltpu.VMEM((ew,), jnp.int32),
            pltpu.VMEM((ew,), jnp.float32),
            pltpu.VMEM((ew,), jnp.int32),
            pltpu.VMEM((ew,), jnp.float32),
            pltpu.VMEM((ew,), jnp.float32),
            pltpu.VMEM((ew,), jnp.float32),
        ],
    )


EH = EP // 2
_geom_h = _make_geom(EH)


BB = 1280


def _msg_body(dist_ref, zs_ref, wrbf_ref, apad_ref, out_ref):
    d = dist_ref[...]
    zs = zs_ref[...]
    centers = lax.broadcasted_iota(jnp.int32, (1, NRBF), 1).astype(jnp.float32) * (
        CUTOFF / (NRBF - 1))
    rbf = jnp.exp(-10.0 * (d - centers) ** 2)
    filt = jnp.dot(rbf, wrbf_ref[...], preferred_element_type=jnp.float32)
    zlane = lax.broadcasted_iota(jnp.int32, (1, D), 1).astype(jnp.float32)
    onehot = jnp.where(zs == zlane, 1.0, 0.0)
    h0 = jnp.dot(onehot, apad_ref[...], preferred_element_type=jnp.float32)
    x = h0 * filt
    out_ref[...] = x * jax.nn.sigmoid(x)


def _msg(distc, zsf, wrbf, apad):
    el = distc.shape[0]
    return pl.pallas_call(
        _msg_body,
        grid=(el // BB,),
        in_specs=[
            pl.BlockSpec((BB, 1), lambda i: (i, 0)),
            pl.BlockSpec((BB, 1), lambda i: (i, 0)),
            pl.BlockSpec((NRBF, D), lambda i: (0, 0)),
            pl.BlockSpec((D, D), lambda i: (0, 0)),
        ],
        out_specs=pl.BlockSpec((BB, D), lambda i: (i, 0)),
        out_shape=jax.ShapeDtypeStruct((el, D), jnp.float32),
    )(distc, zsf, wrbf, apad)


def _scatter_body(el, msg_hbm, dst_hbm, zeros_hbm, aggp_hbm,
                  dstk0_v, dstk1_v, msg0_v, msg1_v, agg_sh, sem0, sem1):
    EW = el // NW
    NCHUNK = EW // K
    cid = lax.axis_index("c")
    sid = lax.axis_index("s")
    pltpu.sync_copy(zeros_hbm.at[pl.ds(sid * NROWS, NROWS)],
                    agg_sh.at[pl.ds(sid * NROWS, NROWS)])
    plsc.subcore_barrier()

    ebase = cid * (el // NC) + sid * EW
    bufs = ((dstk0_v, msg0_v, sem0), (dstk1_v, msg1_v, sem1))

    def start(c, b):
        dstk_v, msg_v, sem = bufs[b]
        pltpu.async_copy(dst_hbm.at[pl.ds(ebase + c * K, K)], dstk_v, sem)
        pltpu.async_copy(msg_hbm.at[pl.ds(ebase + c * K, K)], msg_v, sem)

    def finish(c, b, prefetch):
        dstk_v, msg_v, sem = bufs[b]
        pltpu.make_async_copy(dst_hbm.at[pl.ds(ebase, K)], dstk_v, sem).wait()
        pltpu.make_async_copy(msg_hbm.at[pl.ds(ebase, K)], msg_v, sem).wait()
        pltpu.sync_copy(msg_v, agg_sh.at[dstk_v], add=True)
        if prefetch:
            @pl.when(c + 2 < NCHUNK)
            def _():
                start(c + 2, b)

    start(0, 0)
    start(1, 1)

    def body(i, carry):
        finish(2 * i, 0, True)
        finish(2 * i + 1, 1, True)
        return carry

    lax.fori_loop(0, NCHUNK // 2, body, 0)
    plsc.subcore_barrier()
    pltpu.sync_copy(agg_sh.at[pl.ds(sid * NROWS, NROWS)],
                    aggp_hbm.at[pl.ds(cid * NP + sid * NROWS, NROWS)])


def _make_scatter(el):
    return pl.kernel(
        functools.partial(_scatter_body, el),
        out_type=jax.ShapeDtypeStruct((NC * NP, D), jnp.float32),
        mesh=plsc.VectorSubcoreMesh(core_axis_name="c", subcore_axis_name="s"),
        compiler_params=pltpu.CompilerParams(needs_layout_passes=False),
        scratch_types=[
            pltpu.VMEM((K,), jnp.int32),
            pltpu.VMEM((K,), jnp.int32),
            pltpu.VMEM((K, D), jnp.float32),
            pltpu.VMEM((K, D), jnp.float32),
            pltpu.VMEM_SHARED((NP, D), jnp.float32),
            pltpu.SemaphoreType.DMA,
            pltpu.SemaphoreType.DMA,
        ],
    )


_scatter_h = _make_scatter(EH)


BN = 2048


def _node_body(agg0_ref, agg1_ref, agg2_ref, agg3_ref, z_ref, b_ref,
               apad_ref, wupd_ref, we1_ref, we2_ref, wf1_ref, wf2_ref,
               e_ref, fn_ref):
    i = pl.program_id(0)
    zf = z_ref[...]
    zlane = lax.broadcasted_iota(jnp.int32, (1, D), 1).astype(jnp.float32)
    onehot = jnp.where(zf == zlane, 1.0, 0.0)
    h0 = jnp.dot(onehot, apad_ref[...], preferred_element_type=jnp.float32)
    agg = (agg0_ref[...] + agg1_ref[...]) + (agg2_ref[...] + agg3_ref[...])
    u = jnp.dot(agg, wupd_ref[...], preferred_element_type=jnp.float32)
    h = h0 + u * jax.nn.sigmoid(u)
    e1 = jnp.dot(h, we1_ref[...], preferred_element_type=jnp.float32)
    e1 = e1 * jax.nn.sigmoid(e1)
    ne = jnp.dot(e1, we2_ref[...], preferred_element_type=jnp.float32)
    f1 = jnp.dot(h, wf1_ref[...], preferred_element_type=jnp.float32)
    f1 = f1 * jax.nn.sigmoid(f1)
    fn = jnp.dot(f1, wf2_ref[...], preferred_element_type=jnp.float32)
    fn_ref[...] = fn
    glane = lax.broadcasted_iota(jnp.int32, (1, G), 1).astype(jnp.float32)
    contrib = jnp.where(b_ref[...] == glane, ne, 0.0)
    part = jnp.sum(contrib, axis=0, keepdims=True)

    @pl.when(i == 0)
    def _():
        e_ref[...] = jnp.zeros_like(e_ref)

    e_ref[...] += part


def _node(aggp1, aggp2, zf, bf, apad, wupd, we1, we2, wf1, wf2):
    nb = NP // BN
    return pl.pallas_call(
        _node_body,
        grid=(nb,),
        in_specs=[
            pl.BlockSpec((BN, D), lambda i: (i, 0)),
            pl.BlockSpec((BN, D), lambda i, nb=nb: (i + nb, 0)),
            pl.BlockSpec((BN, D), lambda i: (i, 0)),
            pl.BlockSpec((BN, D), lambda i, nb=nb: (i + nb, 0)),
            pl.BlockSpec((BN, 1), lambda i: (i, 0)),
            pl.BlockSpec((BN, 1), lambda i: (i, 0)),
            pl.BlockSpec((D, D), lambda i: (0, 0)),
            pl.BlockSpec((D, D), lambda i: (0, 0)),
            pl.BlockSpec((D, D), lambda i: (0, 0)),
            pl.BlockSpec((D, 1), lambda i: (0, 0)),
            pl.BlockSpec((D, D), lambda i: (0, 0)),
            pl.BlockSpec((D, 1), lambda i: (0, 0)),
        ],
        out_specs=[
            pl.BlockSpec((1, G), lambda i: (0, 0)),
            pl.BlockSpec((BN, 1), lambda i: (i, 0)),
        ],
        out_shape=[
            jax.ShapeDtypeStruct((1, G), jnp.float32),
            jax.ShapeDtypeStruct((NP, 1), jnp.float32),
        ],
    )(aggp1, aggp1, aggp2, aggp2, zf, bf, apad, wupd, we1, we2, wf1, wf2)


def _force_body(pk_hbm, fn_hbm, zeros_hbm, fp_hbm,
                fn_v, pk0_v, pk1_v, sk0_v, sk1_v, dk0_v, dk1_v,
                fx0_v, fy0_v, fz0_v, fx1_v, fy1_v, fz1_v,
                fxd_sh, fyd_sh, fzd_sh, fxs_sh, fys_sh, fzs_sh,
                semi0, semi1, sems0, sems1):
    cid = lax.axis_index("c")
    sid = lax.axis_index("s")
    rs = sid * NROWS
    for tab in (fxd_sh, fyd_sh, fzd_sh, fxs_sh, fys_sh, fzs_sh):
        pltpu.sync_copy(zeros_hbm.at[pl.ds(rs, NROWS)], tab.at[pl.ds(rs, NROWS)])
    pltpu.sync_copy(fn_hbm, fn_v)
    plsc.subcore_barrier()

    ebase = cid * (EP // NC) + sid * EW
    lane = lax.iota(jnp.int32, 16)
    czero = jnp.zeros((16,), jnp.int32)
    bufs = ((pk0_v, sk0_v, dk0_v, fx0_v, fy0_v, fz0_v, semi0, sems0),
            (pk1_v, sk1_v, dk1_v, fx1_v, fy1_v, fz1_v, semi1, sems1))

    def start_in(c, b):
        pk_v, _, _, _, _, _, semi, _ = bufs[b]
        pltpu.async_copy(pk_hbm.at[pl.ds(ebase + c * KE, KE)], pk_v, semi)

    def scat_pairs(b):
        _, sk_v, dk_v, fx_v, fy_v, fz_v, _, _ = bufs[b]
        return ((fx_v, fxd_sh, dk_v), (fy_v, fyd_sh, dk_v), (fz_v, fzd_sh, dk_v),
                (fx_v, fxs_sh, sk_v), (fy_v, fys_sh, sk_v), (fz_v, fzs_sh, sk_v))

    def process(c, b, first, prefetch):
        pk_v, sk_v, dk_v, fx_v, fy_v, fz_v, semi, sems = bufs[b]
        pltpu.make_async_copy(pk_hbm.at[pl.ds(ebase, KE)], pk_v, semi).wait()
        del first
        for g in range(KE // 16):
            off = g * 16
            rows = off + lane
            s16 = plsc.load_gather(pk_v, [rows, czero])
            d16 = plsc.load_gather(pk_v, [rows, czero + 1])
            ux = plsc.bitcast(plsc.load_gather(pk_v, [rows, czero + 2]), jnp.float32)
            uy = plsc.bitcast(plsc.load_gather(pk_v, [rows, czero + 3]), jnp.float32)
            uz = plsc.bitcast(plsc.load_gather(pk_v, [rows, czero + 4]), jnp.float32)
            fnd = plsc.load_gather(fn_v, [d16])
            sk_v[pl.ds(off, 16)] = s16
            dk_v[pl.ds(off, 16)] = d16
            fx_v[pl.ds(off, 16)] = fnd * ux
            fy_v[pl.ds(off, 16)] = fnd * uy
            fz_v[pl.ds(off, 16)] = fnd * uz
        for val, tab, idx in scat_pairs(b):
            pltpu.sync_copy(val, tab.at[idx], add=True)
        if prefetch:
            @pl.when(c + 2 < NCHUNKE)
            def _():
                start_in(c + 2, b)

    start_in(0, 0)
    start_in(1, 1)
    process(0, 0, True, True)
    process(1, 1, True, True)

    def body(i, carry):
        process(2 * i + 2, 0, False, True)
        process(2 * i + 3, 1, False, True)
        return carry

    lax.fori_loop(0, (NCHUNKE - 2) // 2, body, 0)
    plsc.subcore_barrier()
    fbase = cid * 6 * NP
    for t, tab in enumerate((fxd_sh, fyd_sh, fzd_sh, fxs_sh, fys_sh, fzs_sh)):
        pltpu.sync_copy(tab.at[pl.ds(rs, NROWS)],
                        fp_hbm.at[pl.ds(fbase + t * NP + rs, NROWS)])


_force = pl.kernel(
    _force_body,
    out_type=jax.ShapeDtypeStruct((NC * 6 * NP,), jnp.float32),
    mesh=plsc.VectorSubcoreMesh(core_axis_name="c", subcore_axis_name="s"),
    compiler_params=pltpu.CompilerParams(needs_layout_passes=False),
    scratch_types=[
        pltpu.VMEM((N,), jnp.float32),
        pltpu.VMEM((KE, 8), jnp.int32),
        pltpu.VMEM((KE, 8), jnp.int32),
        pltpu.VMEM((KE,), jnp.int32),
        pltpu.VMEM((KE,), jnp.int32),
        pltpu.VMEM((KE,), jnp.int32),
        pltpu.VMEM((KE,), jnp.int32),
        pltpu.VMEM((KE,), jnp.float32),
        pltpu.VMEM((KE,), jnp.float32),
        pltpu.VMEM((KE,), jnp.float32),
        pltpu.VMEM((KE,), jnp.float32),
        pltpu.VMEM((KE,), jnp.float32),
        pltpu.VMEM((KE,), jnp.float32),
        pltpu.VMEM_SHARED((NP,), jnp.float32),
        pltpu.VMEM_SHARED((NP,), jnp.float32),
        pltpu.VMEM_SHARED((NP,), jnp.float32),
        pltpu.VMEM_SHARED((NP,), jnp.float32),
        pltpu.VMEM_SHARED((NP,), jnp.float32),
        pltpu.VMEM_SHARED((NP,), jnp.float32),
        pltpu.SemaphoreType.DMA,
        pltpu.SemaphoreType.DMA,
        pltpu.SemaphoreType.DMA,
        pltpu.SemaphoreType.DMA,
    ],
)


CB = 2048


def _combine_body(d0_ref, s0_ref, d1_ref, s1_ref, out_ref):
    out_ref[...] = d0_ref[0] + d1_ref[0] - s0_ref[0] - s1_ref[0]


def _combine(fp):
    nb = NP // CB
    return pl.pallas_call(
        _combine_body,
        grid=(nb,),
        in_specs=[
            pl.BlockSpec((1, 3, CB), lambda i: (0, 0, i)),
            pl.BlockSpec((1, 3, CB), lambda i: (1, 0, i)),
            pl.BlockSpec((1, 3, CB), lambda i: (2, 0, i)),
            pl.BlockSpec((1, 3, CB), lambda i: (3, 0, i)),
        ],
        out_specs=pl.BlockSpec((3, CB), lambda i: (0, i)),
        out_shape=jax.ShapeDtypeStruct((3, NP), jnp.float32),
    )(fp, fp, fp, fp)


def kernel(pos, z, batch, edge_index, atom_embed, W_rbf, W_upd, W_e1, w_e2,
           W_f1, w_f2):
    src = edge_index[0].astype(jnp.int32)
    dst = edge_index[1].astype(jnp.int32)
    pad = jnp.zeros((EP - E,), jnp.int32)
    srcp = jnp.concatenate([src, pad])
    dstp = jnp.concatenate([dst, pad])
    dstc = jnp.concatenate([dst, jnp.full((EP - E,), N, jnp.int32)])
    px = jnp.asarray(pos[:, 0], jnp.float32)
    py = jnp.asarray(pos[:, 1], jnp.float32)
    pz = jnp.asarray(pos[:, 2], jnp.float32)
    zi = z.astype(jnp.int32)

    apad = jnp.zeros((D, D), jnp.float32).at[:NZ].set(atom_embed)
    zeros128 = jnp.zeros((NP, D), jnp.float32)

    d1, zs1, ux1, uy1, uz1 = _geom_h(srcp[:EH], dstp[:EH], zi, px, py, pz)
    d2, zs2, ux2, uy2, uz2 = _geom_h(srcp[EH:], dstp[EH:], zi, px, py, pz)

    msg1 = _msg(d1.reshape(EH, 1), zs1.astype(jnp.float32).reshape(EH, 1),
                W_rbf, apad)
    aggp1 = _scatter_h(msg1, dstc[:EH], zeros128)
    msg2 = _msg(d2.reshape(EH, 1), zs2.astype(jnp.float32).reshape(EH, 1),
                W_rbf, apad)
    aggp2 = _scatter_h(msg2, dstc[EH:], zeros128)

    zp = jnp.full((NP, 1), -1.0, jnp.float32).at[:N, 0].set(zi.astype(jnp.float32))
    bp = jnp.full((NP, 1), -1.0, jnp.float32).at[:N, 0].set(batch.astype(jnp.float32))
    energy1, fn = _node(aggp1, aggp2, zp, bp, apad, W_upd, W_e1, w_e2, W_f1,
                        w_f2)

    bc = lax.bitcast_convert_type
    ux = jnp.concatenate([ux1, ux2])
    uy = jnp.concatenate([uy1, uy2])
    uz = jnp.concatenate([uz1, uz2])
    pk8 = jnp.stack([srcp, dstp, bc(ux, jnp.int32), bc(uy, jnp.int32),
                     bc(uz, jnp.int32), pad_col := jnp.zeros((EP,), jnp.int32),
                     pad_col, pad_col], axis=1)

    zerosn = jnp.zeros((NP,), jnp.float32)
    fp = _force(pk8, fn.reshape(NP)[:N], zerosn)

    fsum = _combine(fp.reshape(4, 3, NP))
    return (energy1.reshape(G), fsum.T[:N])


# speedup vs baseline: 1.0255x; 1.0255x over previous
import functools

import jax
import jax.numpy as jnp
from jax import lax
from jax.experimental import pallas as pl
from jax.experimental.pallas import tpu as pltpu
from jax.experimental.pallas import tpu_sc as plsc

N = 10000
E = 320000
D = 128
NRBF = 32
G = 256
NZ = 90
CUTOFF = 6.0

NC = 2
NS = 16
NW = NC * NS
EP = 327680
EW = EP // NW
K = 128
NCHUNK = EW // K
KE = 128
NCHUNKE = EW // KE
NP = 10240
NROWS = NP // NS


def _rsqrt16(x):
    i = lax.bitcast_convert_type(x, jnp.int32)
    i = jnp.int32(0x5F3759DF) - lax.shift_right_arithmetic(i, 1)
    y = lax.bitcast_convert_type(i, jnp.float32)
    for _ in range(3):
        y = y * (1.5 - 0.5 * x * y * y)
    return y


def _geom_body(src_hbm, dst_hbm, z_hbm, px_hbm, py_hbm, pz_hbm,
               dist_hbm, zsrc_hbm, ux_hbm, uy_hbm, uz_hbm,
               px_v, py_v, pz_v, z_v, src_v, dst_v, dist_v, zs_v,
               ux_v, uy_v, uz_v):
    cid = lax.axis_index("c")
    sid = lax.axis_index("s")
    wid = sid * NC + cid
    base = wid * EW

    pltpu.sync_copy(px_hbm, px_v)
    pltpu.sync_copy(py_hbm, py_v)
    pltpu.sync_copy(pz_hbm, pz_v)
    pltpu.sync_copy(z_hbm, z_v)
    pltpu.sync_copy(src_hbm.at[pl.ds(base, EW)], src_v)
    pltpu.sync_copy(dst_hbm.at[pl.ds(base, EW)], dst_v)

    def body(g, carry):
        off = g * 16
        s16 = src_v[pl.ds(off, 16)]
        d16 = dst_v[pl.ds(off, 16)]
        dx = plsc.load_gather(px_v, [s16]) - plsc.load_gather(px_v, [d16])
        dy = plsc.load_gather(py_v, [s16]) - plsc.load_gather(py_v, [d16])
        dz = plsc.load_gather(pz_v, [s16]) - plsc.load_gather(pz_v, [d16])
        zx = plsc.load_gather(z_v, [s16])
        sq = dx * dx + dy * dy + dz * dz + 1e-8
        r = _rsqrt16(sq)
        dist_v[pl.ds(off, 16)] = sq * r
        zs_v[pl.ds(off, 16)] = zx
        ux_v[pl.ds(off, 16)] = dx * r
        uy_v[pl.ds(off, 16)] = dy * r
        uz_v[pl.ds(off, 16)] = dz * r
        return carry

    lax.fori_loop(0, EW // 16, body, 0)
    pltpu.sync_copy(dist_v, dist_hbm.at[pl.ds(base, EW)])
    pltpu.sync_copy(zs_v, zsrc_hbm.at[pl.ds(base, EW)])
    pltpu.sync_copy(ux_v, ux_hbm.at[pl.ds(base, EW)])
    pltpu.sync_copy(uy_v, uy_hbm.at[pl.ds(base, EW)])
    pltpu.sync_copy(uz_v, uz_hbm.at[pl.ds(base, EW)])


_geom = pl.kernel(
    _geom_body,
    out_type=(jax.ShapeDtypeStruct((EP,), jnp.float32),
              jax.ShapeDtypeStruct((EP,), jnp.int32),
              jax.ShapeDtypeStruct((EP,), jnp.float32),
              jax.ShapeDtypeStruct((EP,), jnp.float32),
              jax.ShapeDtypeStruct((EP,), jnp.float32)),
    mesh=plsc.VectorSubcoreMesh(core_axis_name="c", subcore_axis_name="s"),
    compiler_params=pltpu.CompilerParams(needs_layout_passes=False),
    scratch_types=[
        pltpu.VMEM((N,), jnp.float32),
        pltpu.VMEM((N,), jnp.float32),
        pltpu.VMEM((N,), jnp.float32),
        pltpu.VMEM((N,), jnp.int32),
        pltpu.VMEM((EW,), jnp.int32),
        pltpu.VMEM((EW,), jnp.int32),
        pltpu.VMEM((EW,), jnp.float32),
        pltpu.VMEM((EW,), jnp.int32),
        pltpu.VMEM((EW,), jnp.float32),
        pltpu.VMEM((EW,), jnp.float32),
        pltpu.VMEM((EW,), jnp.float32),
    ],
)


BB = 1280


def _msg_body(dist_ref, zs_ref, wrbf_ref, apad_ref, out_ref):
    d = dist_ref[...]
    zs = zs_ref[...]
    centers = lax.broadcasted_iota(jnp.int32, (1, NRBF), 1).astype(jnp.float32) * (
        CUTOFF / (NRBF - 1))
    rbf = jnp.exp(-10.0 * (d - centers) ** 2)
    filt = jnp.dot(rbf, wrbf_ref[...], preferred_element_type=jnp.float32)
    zlane = lax.broadcasted_iota(jnp.int32, (1, D), 1).astype(jnp.float32)
    onehot = jnp.where(zs == zlane, 1.0, 0.0)
    h0 = jnp.dot(onehot, apad_ref[...], preferred_element_type=jnp.float32)
    x = h0 * filt
    out_ref[...] = x * jax.nn.sigmoid(x)


def _msg(distc, zsf, wrbf, apad):
    return pl.pallas_call(
        _msg_body,
        grid=(EP // BB,),
        in_specs=[
            pl.BlockSpec((BB, 1), lambda i: (i, 0)),
            pl.BlockSpec((BB, 1), lambda i: (i, 0)),
            pl.BlockSpec((NRBF, D), lambda i: (0, 0)),
            pl.BlockSpec((D, D), lambda i: (0, 0)),
        ],
        out_specs=pl.BlockSpec((BB, D), lambda i: (i, 0)),
        out_shape=jax.ShapeDtypeStruct((EP, D), jnp.float32),
    )(distc, zsf, wrbf, apad)


def _scatter_body(msg_hbm, dst_hbm, zeros_hbm, aggp_hbm,
                  dstk0_v, dstk1_v, msg0_v, msg1_v, agg_sh, sem0, sem1):
    cid = lax.axis_index("c")
    sid = lax.axis_index("s")
    pltpu.sync_copy(zeros_hbm.at[pl.ds(sid * NROWS, NROWS)],
                    agg_sh.at[pl.ds(sid * NROWS, NROWS)])
    plsc.subcore_barrier()

    ebase = cid * (EP // NC) + sid * EW
    bufs = ((dstk0_v, msg0_v, sem0), (dstk1_v, msg1_v, sem1))

    def start(c, b):
        dstk_v, msg_v, sem = bufs[b]
        pltpu.async_copy(dst_hbm.at[pl.ds(ebase + c * K, K)], dstk_v, sem)
        pltpu.async_copy(msg_hbm.at[pl.ds(ebase + c * K, K)], msg_v, sem)

    def finish(c, b, prefetch):
        dstk_v, msg_v, sem = bufs[b]
        pltpu.make_async_copy(dst_hbm.at[pl.ds(ebase, K)], dstk_v, sem).wait()
        pltpu.make_async_copy(msg_hbm.at[pl.ds(ebase, K)], msg_v, sem).wait()
        pltpu.sync_copy(msg_v, agg_sh.at[dstk_v], add=True)
        if prefetch:
            @pl.when(c + 2 < NCHUNK)
            def _():
                start(c + 2, b)

    start(0, 0)
    start(1, 1)

    def body(i, carry):
        finish(2 * i, 0, True)
        finish(2 * i + 1, 1, True)
        return carry

    lax.fori_loop(0, NCHUNK // 2, body, 0)
    plsc.subcore_barrier()
    pltpu.sync_copy(agg_sh.at[pl.ds(sid * NROWS, NROWS)],
                    aggp_hbm.at[pl.ds(cid * NP + sid * NROWS, NROWS)])


_scatter = pl.kernel(
    _scatter_body,
    out_type=jax.ShapeDtypeStruct((NC * NP, D), jnp.float32),
    mesh=plsc.VectorSubcoreMesh(core_axis_name="c", subcore_axis_name="s"),
    compiler_params=pltpu.CompilerParams(needs_layout_passes=False),
    scratch_types=[
        pltpu.VMEM((K,), jnp.int32),
        pltpu.VMEM((K,), jnp.int32),
        pltpu.VMEM((K, D), jnp.float32),
        pltpu.VMEM((K, D), jnp.float32),
        pltpu.VMEM_SHARED((NP, D), jnp.float32),
        pltpu.SemaphoreType.DMA,
        pltpu.SemaphoreType.DMA,
    ],
)


BN = 2048


def _node_body(agg0_ref, agg1_ref, z_ref, b_ref, apad_ref, wupd_ref,
               we1_ref, we2_ref, wf1_ref, wf2_ref, e_ref, fn_ref):
    i = pl.program_id(0)
    zf = z_ref[...]
    zlane = lax.broadcasted_iota(jnp.int32, (1, D), 1).astype(jnp.float32)
    onehot = jnp.where(zf == zlane, 1.0, 0.0)
    h0 = jnp.dot(onehot, apad_ref[...], preferred_element_type=jnp.float32)
    agg = agg0_ref[...] + agg1_ref[...]
    u = jnp.dot(agg, wupd_ref[...], preferred_element_type=jnp.float32)
    h = h0 + u * jax.nn.sigmoid(u)
    e1 = jnp.dot(h, we1_ref[...], preferred_element_type=jnp.float32)
    e1 = e1 * jax.nn.sigmoid(e1)
    ne = jnp.dot(e1, we2_ref[...], preferred_element_type=jnp.float32)
    f1 = jnp.dot(h, wf1_ref[...], preferred_element_type=jnp.float32)
    f1 = f1 * jax.nn.sigmoid(f1)
    fn = jnp.dot(f1, wf2_ref[...], preferred_element_type=jnp.float32)
    fn_ref[...] = fn
    glane = lax.broadcasted_iota(jnp.int32, (1, G), 1).astype(jnp.float32)
    contrib = jnp.where(b_ref[...] == glane, ne, 0.0)
    part = jnp.sum(contrib, axis=0, keepdims=True)

    @pl.when(i == 0)
    def _():
        e_ref[...] = jnp.zeros_like(e_ref)

    e_ref[...] += part


def _node(aggp, zf, bf, apad, wupd, we1, we2, wf1, wf2):
    nb = NP // BN
    return pl.pallas_call(
        _node_body,
        grid=(nb,),
        in_specs=[
            pl.BlockSpec((BN, D), lambda i: (i, 0)),
            pl.BlockSpec((BN, D), lambda i, nb=nb: (i + nb, 0)),
            pl.BlockSpec((BN, 1), lambda i: (i, 0)),
            pl.BlockSpec((BN, 1), lambda i: (i, 0)),
            pl.BlockSpec((D, D), lambda i: (0, 0)),
            pl.BlockSpec((D, D), lambda i: (0, 0)),
            pl.BlockSpec((D, D), lambda i: (0, 0)),
            pl.BlockSpec((D, 1), lambda i: (0, 0)),
            pl.BlockSpec((D, D), lambda i: (0, 0)),
            pl.BlockSpec((D, 1), lambda i: (0, 0)),
        ],
        out_specs=[
            pl.BlockSpec((1, G), lambda i: (0, 0)),
            pl.BlockSpec((BN, 1), lambda i: (i, 0)),
        ],
        out_shape=[
            jax.ShapeDtypeStruct((1, G), jnp.float32),
            jax.ShapeDtypeStruct((NP, 1), jnp.float32),
        ],
    )(aggp, aggp, zf, bf, apad, wupd, we1, we2, wf1, wf2)


def _force_body(pk_hbm, fn_hbm, zeros_hbm, fp_hbm,
                fn_v, pk0_v, pk1_v, sk0_v, sk1_v, dk0_v, dk1_v,
                fx0_v, fy0_v, fz0_v, fx1_v, fy1_v, fz1_v,
                fxd_sh, fyd_sh, fzd_sh, fxs_sh, fys_sh, fzs_sh,
                semi0, semi1, sems0, sems1):
    cid = lax.axis_index("c")
    sid = lax.axis_index("s")
    rs = sid * NROWS
    for tab in (fxd_sh, fyd_sh, fzd_sh, fxs_sh, fys_sh, fzs_sh):
        pltpu.sync_copy(zeros_hbm.at[pl.ds(rs, NROWS)], tab.at[pl.ds(rs, NROWS)])
    pltpu.sync_copy(fn_hbm, fn_v)
    plsc.subcore_barrier()

    ebase = cid * (EP // NC) + sid * EW
    lane = lax.iota(jnp.int32, 16)
    czero = jnp.zeros((16,), jnp.int32)
    bufs = ((pk0_v, sk0_v, dk0_v, fx0_v, fy0_v, fz0_v, semi0, sems0),
            (pk1_v, sk1_v, dk1_v, fx1_v, fy1_v, fz1_v, semi1, sems1))

    def start_in(c, b):
        pk_v, _, _, _, _, _, semi, _ = bufs[b]
        pltpu.async_copy(pk_hbm.at[pl.ds(ebase + c * KE, KE)], pk_v, semi)

    def scat_pairs(b):
        _, sk_v, dk_v, fx_v, fy_v, fz_v, _, _ = bufs[b]
        return ((fx_v, fxd_sh, dk_v), (fy_v, fyd_sh, dk_v), (fz_v, fzd_sh, dk_v),
                (fx_v, fxs_sh, sk_v), (fy_v, fys_sh, sk_v), (fz_v, fzs_sh, sk_v))

    def process(c, b, first, prefetch):
        pk_v, sk_v, dk_v, fx_v, fy_v, fz_v, semi, sems = bufs[b]
        pltpu.make_async_copy(pk_hbm.at[pl.ds(ebase, KE)], pk_v, semi).wait()
        del first
        for g in range(KE // 16):
            off = g * 16
            rows = off + lane
            s16 = plsc.load_gather(pk_v, [rows, czero])
            d16 = plsc.load_gather(pk_v, [rows, czero + 1])
            ux = plsc.bitcast(plsc.load_gather(pk_v, [rows, czero + 2]), jnp.float32)
            uy = plsc.bitcast(plsc.load_gather(pk_v, [rows, czero + 3]), jnp.float32)
            uz = plsc.bitcast(plsc.load_gather(pk_v, [rows, czero + 4]), jnp.float32)
            fnd = plsc.load_gather(fn_v, [d16])
            sk_v[pl.ds(off, 16)] = s16
            dk_v[pl.ds(off, 16)] = d16
            fx_v[pl.ds(off, 16)] = fnd * ux
            fy_v[pl.ds(off, 16)] = fnd * uy
            fz_v[pl.ds(off, 16)] = fnd * uz
        for val, tab, idx in scat_pairs(b):
            pltpu.sync_copy(val, tab.at[idx], add=True)
        if prefetch:
            @pl.when(c + 2 < NCHUNKE)
            def _():
                start_in(c + 2, b)

    start_in(0, 0)
    start_in(1, 1)
    process(0, 0, True, True)
    process(1, 1, True, True)

    def body(i, carry):
        process(2 * i + 2, 0, False, True)
        process(2 * i + 3, 1, False, True)
        return carry

    lax.fori_loop(0, (NCHUNKE - 2) // 2, body, 0)
    plsc.subcore_barrier()
    fbase = cid * 6 * NP
    for t, tab in enumerate((fxd_sh, fyd_sh, fzd_sh, fxs_sh, fys_sh, fzs_sh)):
        pltpu.sync_copy(tab.at[pl.ds(rs, NROWS)],
                        fp_hbm.at[pl.ds(fbase + t * NP + rs, NROWS)])


_force = pl.kernel(
    _force_body,
    out_type=jax.ShapeDtypeStruct((NC * 6 * NP,), jnp.float32),
    mesh=plsc.VectorSubcoreMesh(core_axis_name="c", subcore_axis_name="s"),
    compiler_params=pltpu.CompilerParams(needs_layout_passes=False),
    scratch_types=[
        pltpu.VMEM((N,), jnp.float32),
        pltpu.VMEM((KE, 8), jnp.int32),
        pltpu.VMEM((KE, 8), jnp.int32),
        pltpu.VMEM((KE,), jnp.int32),
        pltpu.VMEM((KE,), jnp.int32),
        pltpu.VMEM((KE,), jnp.int32),
        pltpu.VMEM((KE,), jnp.int32),
        pltpu.VMEM((KE,), jnp.float32),
        pltpu.VMEM((KE,), jnp.float32),
        pltpu.VMEM((KE,), jnp.float32),
        pltpu.VMEM((KE,), jnp.float32),
        pltpu.VMEM((KE,), jnp.float32),
        pltpu.VMEM((KE,), jnp.float32),
        pltpu.VMEM_SHARED((NP,), jnp.float32),
        pltpu.VMEM_SHARED((NP,), jnp.float32),
        pltpu.VMEM_SHARED((NP,), jnp.float32),
        pltpu.VMEM_SHARED((NP,), jnp.float32),
        pltpu.VMEM_SHARED((NP,), jnp.float32),
        pltpu.VMEM_SHARED((NP,), jnp.float32),
        pltpu.SemaphoreType.DMA,
        pltpu.SemaphoreType.DMA,
        pltpu.SemaphoreType.DMA,
        pltpu.SemaphoreType.DMA,
    ],
)


CB = 2048


def _combine_body(d0_ref, s0_ref, d1_ref, s1_ref, out_ref):
    out_ref[...] = d0_ref[0] + d1_ref[0] - s0_ref[0] - s1_ref[0]


def _combine(fp):
    nb = NP // CB
    return pl.pallas_call(
        _combine_body,
        grid=(nb,),
        in_specs=[
            pl.BlockSpec((1, 3, CB), lambda i: (0, 0, i)),
            pl.BlockSpec((1, 3, CB), lambda i: (1, 0, i)),
            pl.BlockSpec((1, 3, CB), lambda i: (2, 0, i)),
            pl.BlockSpec((1, 3, CB), lambda i: (3, 0, i)),
        ],
        out_specs=pl.BlockSpec((3, CB), lambda i: (0, i)),
        out_shape=jax.ShapeDtypeStruct((3, NP), jnp.float32),
    )(fp, fp, fp, fp)


def kernel(pos, z, batch, edge_index, atom_embed, W_rbf, W_upd, W_e1, w_e2,
           W_f1, w_f2):
    src = edge_index[0].astype(jnp.int32)
    dst = edge_index[1].astype(jnp.int32)
    pad = jnp.zeros((EP - E,), jnp.int32)
    srcp = jnp.concatenate([src, pad])
    dstp = jnp.concatenate([dst, pad])
    dstc = jnp.concatenate([dst, jnp.full((EP - E,), N, jnp.int32)])
    px = jnp.asarray(pos[:, 0], jnp.float32)
    py = jnp.asarray(pos[:, 1], jnp.float32)
    pz = jnp.asarray(pos[:, 2], jnp.float32)
    zi = z.astype(jnp.int32)

    dist, zsrc, ux, uy, uz = _geom(srcp, dstp, zi, px, py, pz)

    apad = jnp.zeros((D, D), jnp.float32).at[:NZ].set(atom_embed)
    msg = _msg(dist.reshape(EP, 1), zsrc.astype(jnp.float32).reshape(EP, 1),
               W_rbf, apad)

    zeros128 = jnp.zeros((NP, D), jnp.float32)
    aggp = _scatter(msg, dstc, zeros128)

    zp = jnp.full((NP, 1), -1.0, jnp.float32).at[:N, 0].set(zi.astype(jnp.float32))
    bp = jnp.full((NP, 1), -1.0, jnp.float32).at[:N, 0].set(batch.astype(jnp.float32))
    energy1, fn = _node(aggp, zp, bp, apad, W_upd, W_e1, w_e2, W_f1, w_f2)

    bc = lax.bitcast_convert_type
    pk8 = jnp.stack([srcp, dstp, bc(ux, jnp.int32), bc(uy, jnp.int32),
                     bc(uz, jnp.int32), pad_col := jnp.zeros((EP,), jnp.int32),
                     pad_col, pad_col], axis=1)

    zerosn = jnp.zeros((NP,), jnp.float32)
    fp = _force(pk8, fn.reshape(NP)[:N], zerosn)

    fsum = _combine(fp.reshape(4, 3, NP))
    return (energy1.reshape(G), fsum.T[:N])


# speedup vs baseline: 1.1458x; 1.1173x over previous
import functools

import jax
import jax.numpy as jnp
from jax import lax
from jax.experimental import pallas as pl
from jax.experimental.pallas import tpu as pltpu
from jax.experimental.pallas import tpu_sc as plsc

N = 10000
E = 320000
D = 128
NRBF = 32
G = 256
NZ = 90
CUTOFF = 6.0

NC = 2
NS = 16
NW = NC * NS
EP = 327680
EW = EP // NW
K = 128
NCHUNK = EW // K
KE = 128
NCHUNKE = EW // KE
NP = 10240
NROWS = NP // NS


def _rsqrt16(x):
    i = lax.bitcast_convert_type(x, jnp.int32)
    i = jnp.int32(0x5F3759DF) - lax.shift_right_arithmetic(i, 1)
    y = lax.bitcast_convert_type(i, jnp.float32)
    for _ in range(3):
        y = y * (1.5 - 0.5 * x * y * y)
    return y


def _geom_body(src_hbm, dst_hbm, z_hbm, px_hbm, py_hbm, pz_hbm,
               dist_hbm, zsrc_hbm, ux_hbm, uy_hbm, uz_hbm,
               px_v, py_v, pz_v, z_v, src_v, dst_v, dist_v, zs_v,
               ux_v, uy_v, uz_v):
    cid = lax.axis_index("c")
    sid = lax.axis_index("s")
    wid = sid * NC + cid
    base = wid * EW

    pltpu.sync_copy(px_hbm, px_v)
    pltpu.sync_copy(py_hbm, py_v)
    pltpu.sync_copy(pz_hbm, pz_v)
    pltpu.sync_copy(z_hbm, z_v)
    pltpu.sync_copy(src_hbm.at[pl.ds(base, EW)], src_v)
    pltpu.sync_copy(dst_hbm.at[pl.ds(base, EW)], dst_v)

    def body(g, carry):
        off = g * 16
        s16 = src_v[pl.ds(off, 16)]
        d16 = dst_v[pl.ds(off, 16)]
        dx = plsc.load_gather(px_v, [s16]) - plsc.load_gather(px_v, [d16])
        dy = plsc.load_gather(py_v, [s16]) - plsc.load_gather(py_v, [d16])
        dz = plsc.load_gather(pz_v, [s16]) - plsc.load_gather(pz_v, [d16])
        zx = plsc.load_gather(z_v, [s16])
        sq = dx * dx + dy * dy + dz * dz + 1e-8
        r = _rsqrt16(sq)
        dist_v[pl.ds(off, 16)] = sq * r
        zs_v[pl.ds(off, 16)] = zx
        ux_v[pl.ds(off, 16)] = dx * r
        uy_v[pl.ds(off, 16)] = dy * r
        uz_v[pl.ds(off, 16)] = dz * r
        return carry

    lax.fori_loop(0, EW // 16, body, 0)
    pltpu.sync_copy(dist_v, dist_hbm.at[pl.ds(base, EW)])
    pltpu.sync_copy(zs_v, zsrc_hbm.at[pl.ds(base, EW)])
    pltpu.sync_copy(ux_v, ux_hbm.at[pl.ds(base, EW)])
    pltpu.sync_copy(uy_v, uy_hbm.at[pl.ds(base, EW)])
    pltpu.sync_copy(uz_v, uz_hbm.at[pl.ds(base, EW)])


_geom = pl.kernel(
    _geom_body,
    out_type=(jax.ShapeDtypeStruct((EP,), jnp.float32),
              jax.ShapeDtypeStruct((EP,), jnp.int32),
              jax.ShapeDtypeStruct((EP,), jnp.float32),
              jax.ShapeDtypeStruct((EP,), jnp.float32),
              jax.ShapeDtypeStruct((EP,), jnp.float32)),
    mesh=plsc.VectorSubcoreMesh(core_axis_name="c", subcore_axis_name="s"),
    compiler_params=pltpu.CompilerParams(needs_layout_passes=False),
    scratch_types=[
        pltpu.VMEM((N,), jnp.float32),
        pltpu.VMEM((N,), jnp.float32),
        pltpu.VMEM((N,), jnp.float32),
        pltpu.VMEM((N,), jnp.int32),
        pltpu.VMEM((EW,), jnp.int32),
        pltpu.VMEM((EW,), jnp.int32),
        pltpu.VMEM((EW,), jnp.float32),
        pltpu.VMEM((EW,), jnp.int32),
        pltpu.VMEM((EW,), jnp.float32),
        pltpu.VMEM((EW,), jnp.float32),
        pltpu.VMEM((EW,), jnp.float32),
    ],
)


BB = 2560


def _msg_body(dist_ref, zs_ref, wrbf_ref, apad_ref, out_ref):
    d = dist_ref[...]
    zs = zs_ref[...]
    centers = lax.broadcasted_iota(jnp.int32, (1, NRBF), 1).astype(jnp.float32) * (
        CUTOFF / (NRBF - 1))
    rbf = jnp.exp(-10.0 * (d - centers) ** 2)
    filt = jnp.dot(rbf, wrbf_ref[...], preferred_element_type=jnp.float32)
    zlane = lax.broadcasted_iota(jnp.int32, (1, D), 1).astype(jnp.float32)
    onehot = jnp.where(zs == zlane, 1.0, 0.0)
    h0 = jnp.dot(onehot, apad_ref[...], preferred_element_type=jnp.float32)
    x = h0 * filt
    out_ref[...] = x * jax.nn.sigmoid(x)


def _msg(distc, zsf, wrbf, apad):
    return pl.pallas_call(
        _msg_body,
        grid=(EP // BB,),
        in_specs=[
            pl.BlockSpec((BB, 1), lambda i: (i, 0)),
            pl.BlockSpec((BB, 1), lambda i: (i, 0)),
            pl.BlockSpec((NRBF, D), lambda i: (0, 0)),
            pl.BlockSpec((D, D), lambda i: (0, 0)),
        ],
        out_specs=pl.BlockSpec((BB, D), lambda i: (i, 0)),
        out_shape=jax.ShapeDtypeStruct((EP, D), jnp.float32),
    )(distc, zsf, wrbf, apad)


def _scatter_body(msg_hbm, dst_hbm, zeros_hbm, aggp_hbm,
                  dstk0_v, dstk1_v, msg0_v, msg1_v, agg_sh, sem0, sem1):
    cid = lax.axis_index("c")
    sid = lax.axis_index("s")
    pltpu.sync_copy(zeros_hbm.at[pl.ds(sid * NROWS, NROWS)],
                    agg_sh.at[pl.ds(sid * NROWS, NROWS)])
    plsc.subcore_barrier()

    ebase = cid * (EP // NC) + sid * EW
    bufs = ((dstk0_v, msg0_v, sem0), (dstk1_v, msg1_v, sem1))

    def start(c, b):
        dstk_v, msg_v, sem = bufs[b]
        pltpu.async_copy(dst_hbm.at[pl.ds(ebase + c * K, K)], dstk_v, sem)
        pltpu.async_copy(msg_hbm.at[pl.ds(ebase + c * K, K)], msg_v, sem)

    def finish(c, b, prefetch):
        dstk_v, msg_v, sem = bufs[b]
        pltpu.make_async_copy(dst_hbm.at[pl.ds(ebase, K)], dstk_v, sem).wait()
        pltpu.make_async_copy(msg_hbm.at[pl.ds(ebase, K)], msg_v, sem).wait()
        pltpu.sync_copy(msg_v, agg_sh.at[dstk_v], add=True)
        if prefetch:
            @pl.when(c + 2 < NCHUNK)
            def _():
                start(c + 2, b)

    start(0, 0)
    start(1, 1)

    def body(i, carry):
        finish(2 * i, 0, True)
        finish(2 * i + 1, 1, True)
        return carry

    lax.fori_loop(0, NCHUNK // 2, body, 0)
    plsc.subcore_barrier()
    pltpu.sync_copy(agg_sh.at[pl.ds(sid * NROWS, NROWS)],
                    aggp_hbm.at[pl.ds(cid * NP + sid * NROWS, NROWS)])


_scatter = pl.kernel(
    _scatter_body,
    out_type=jax.ShapeDtypeStruct((NC * NP, D), jnp.float32),
    mesh=plsc.VectorSubcoreMesh(core_axis_name="c", subcore_axis_name="s"),
    compiler_params=pltpu.CompilerParams(needs_layout_passes=False),
    scratch_types=[
        pltpu.VMEM((K,), jnp.int32),
        pltpu.VMEM((K,), jnp.int32),
        pltpu.VMEM((K, D), jnp.float32),
        pltpu.VMEM((K, D), jnp.float32),
        pltpu.VMEM_SHARED((NP, D), jnp.float32),
        pltpu.SemaphoreType.DMA,
        pltpu.SemaphoreType.DMA,
    ],
)


BN = 2048


def _node_body(agg0_ref, agg1_ref, z_ref, b_ref, apad_ref, wupd_ref,
               we1_ref, we2_ref, wf1_ref, wf2_ref, e_ref, fn_ref):
    i = pl.program_id(0)
    zf = z_ref[...]
    zlane = lax.broadcasted_iota(jnp.int32, (1, D), 1).astype(jnp.float32)
    onehot = jnp.where(zf == zlane, 1.0, 0.0)
    h0 = jnp.dot(onehot, apad_ref[...], preferred_element_type=jnp.float32)
    agg = agg0_ref[...] + agg1_ref[...]
    u = jnp.dot(agg, wupd_ref[...], preferred_element_type=jnp.float32)
    h = h0 + u * jax.nn.sigmoid(u)
    e1 = jnp.dot(h, we1_ref[...], preferred_element_type=jnp.float32)
    e1 = e1 * jax.nn.sigmoid(e1)
    ne = jnp.dot(e1, we2_ref[...], preferred_element_type=jnp.float32)
    f1 = jnp.dot(h, wf1_ref[...], preferred_element_type=jnp.float32)
    f1 = f1 * jax.nn.sigmoid(f1)
    fn = jnp.dot(f1, wf2_ref[...], preferred_element_type=jnp.float32)
    fn_ref[...] = fn
    glane = lax.broadcasted_iota(jnp.int32, (1, G), 1).astype(jnp.float32)
    contrib = jnp.where(b_ref[...] == glane, ne, 0.0)
    part = jnp.sum(contrib, axis=0, keepdims=True)

    @pl.when(i == 0)
    def _():
        e_ref[...] = jnp.zeros_like(e_ref)

    e_ref[...] += part


def _node(aggp, zf, bf, apad, wupd, we1, we2, wf1, wf2):
    nb = NP // BN
    return pl.pallas_call(
        _node_body,
        grid=(nb,),
        in_specs=[
            pl.BlockSpec((BN, D), lambda i: (i, 0)),
            pl.BlockSpec((BN, D), lambda i, nb=nb: (i + nb, 0)),
            pl.BlockSpec((BN, 1), lambda i: (i, 0)),
            pl.BlockSpec((BN, 1), lambda i: (i, 0)),
            pl.BlockSpec((D, D), lambda i: (0, 0)),
            pl.BlockSpec((D, D), lambda i: (0, 0)),
            pl.BlockSpec((D, D), lambda i: (0, 0)),
            pl.BlockSpec((D, 1), lambda i: (0, 0)),
            pl.BlockSpec((D, D), lambda i: (0, 0)),
            pl.BlockSpec((D, 1), lambda i: (0, 0)),
        ],
        out_specs=[
            pl.BlockSpec((1, G), lambda i: (0, 0)),
            pl.BlockSpec((BN, 1), lambda i: (i, 0)),
        ],
        out_shape=[
            jax.ShapeDtypeStruct((1, G), jnp.float32),
            jax.ShapeDtypeStruct((NP, 1), jnp.float32),
        ],
    )(aggp, aggp, zf, bf, apad, wupd, we1, we2, wf1, wf2)


def _force_body(pk_hbm, fn_hbm, zeros_hbm, fp_hbm,
                fn_v, pk0_v, pk1_v, sk0_v, sk1_v, dk0_v, dk1_v,
                fx0_v, fy0_v, fz0_v, fx1_v, fy1_v, fz1_v,
                fxd_sh, fyd_sh, fzd_sh, fxs_sh, fys_sh, fzs_sh,
                semi0, semi1, sems0, sems1):
    cid = lax.axis_index("c")
    sid = lax.axis_index("s")
    rs = sid * NROWS
    for tab in (fxd_sh, fyd_sh, fzd_sh, fxs_sh, fys_sh, fzs_sh):
        pltpu.sync_copy(zeros_hbm.at[pl.ds(rs, NROWS)], tab.at[pl.ds(rs, NROWS)])
    pltpu.sync_copy(fn_hbm, fn_v)
    plsc.subcore_barrier()

    ebase = cid * (EP // NC) + sid * EW
    lane = lax.iota(jnp.int32, 16)
    czero = jnp.zeros((16,), jnp.int32)
    bufs = ((pk0_v, sk0_v, dk0_v, fx0_v, fy0_v, fz0_v, semi0, sems0),
            (pk1_v, sk1_v, dk1_v, fx1_v, fy1_v, fz1_v, semi1, sems1))

    def start_in(c, b):
        pk_v, _, _, _, _, _, semi, _ = bufs[b]
        pltpu.async_copy(pk_hbm.at[pl.ds(ebase + c * KE, KE)], pk_v, semi)

    def scat_pairs(b):
        _, sk_v, dk_v, fx_v, fy_v, fz_v, _, _ = bufs[b]
        return ((fx_v, fxd_sh, dk_v), (fy_v, fyd_sh, dk_v), (fz_v, fzd_sh, dk_v),
                (fx_v, fxs_sh, sk_v), (fy_v, fys_sh, sk_v), (fz_v, fzs_sh, sk_v))

    def process(c, b, first, prefetch):
        pk_v, sk_v, dk_v, fx_v, fy_v, fz_v, semi, sems = bufs[b]
        pltpu.make_async_copy(pk_hbm.at[pl.ds(ebase, KE)], pk_v, semi).wait()
        del first
        for g in range(KE // 16):
            off = g * 16
            rows = off + lane
            s16 = plsc.load_gather(pk_v, [rows, czero])
            d16 = plsc.load_gather(pk_v, [rows, czero + 1])
            ux = plsc.bitcast(plsc.load_gather(pk_v, [rows, czero + 2]), jnp.float32)
            uy = plsc.bitcast(plsc.load_gather(pk_v, [rows, czero + 3]), jnp.float32)
            uz = plsc.bitcast(plsc.load_gather(pk_v, [rows, czero + 4]), jnp.float32)
            fnd = plsc.load_gather(fn_v, [d16])
            sk_v[pl.ds(off, 16)] = s16
            dk_v[pl.ds(off, 16)] = d16
            fx_v[pl.ds(off, 16)] = fnd * ux
            fy_v[pl.ds(off, 16)] = fnd * uy
            fz_v[pl.ds(off, 16)] = fnd * uz
        for val, tab, idx in scat_pairs(b):
            pltpu.sync_copy(val, tab.at[idx], add=True)
        if prefetch:
            @pl.when(c + 2 < NCHUNKE)
            def _():
                start_in(c + 2, b)

    start_in(0, 0)
    start_in(1, 1)
    process(0, 0, True, True)
    process(1, 1, True, True)

    def body(i, carry):
        process(2 * i + 2, 0, False, True)
        process(2 * i + 3, 1, False, True)
        return carry

    lax.fori_loop(0, (NCHUNKE - 2) // 2, body, 0)
    plsc.subcore_barrier()
    fbase = cid * 6 * NP
    for t, tab in enumerate((fxd_sh, fyd_sh, fzd_sh, fxs_sh, fys_sh, fzs_sh)):
        pltpu.sync_copy(tab.at[pl.ds(rs, NROWS)],
                        fp_hbm.at[pl.ds(fbase + t * NP + rs, NROWS)])


_force = pl.kernel(
    _force_body,
    out_type=jax.ShapeDtypeStruct((NC * 6 * NP,), jnp.float32),
    mesh=plsc.VectorSubcoreMesh(core_axis_name="c", subcore_axis_name="s"),
    compiler_params=pltpu.CompilerParams(needs_layout_passes=False),
    scratch_types=[
        pltpu.VMEM((N,), jnp.float32),
        pltpu.VMEM((KE, 8), jnp.int32),
        pltpu.VMEM((KE, 8), jnp.int32),
        pltpu.VMEM((KE,), jnp.int32),
        pltpu.VMEM((KE,), jnp.int32),
        pltpu.VMEM((KE,), jnp.int32),
        pltpu.VMEM((KE,), jnp.int32),
        pltpu.VMEM((KE,), jnp.float32),
        pltpu.VMEM((KE,), jnp.float32),
        pltpu.VMEM((KE,), jnp.float32),
        pltpu.VMEM((KE,), jnp.float32),
        pltpu.VMEM((KE,), jnp.float32),
        pltpu.VMEM((KE,), jnp.float32),
        pltpu.VMEM_SHARED((NP,), jnp.float32),
        pltpu.VMEM_SHARED((NP,), jnp.float32),
        pltpu.VMEM_SHARED((NP,), jnp.float32),
        pltpu.VMEM_SHARED((NP,), jnp.float32),
        pltpu.VMEM_SHARED((NP,), jnp.float32),
        pltpu.VMEM_SHARED((NP,), jnp.float32),
        pltpu.SemaphoreType.DMA,
        pltpu.SemaphoreType.DMA,
        pltpu.SemaphoreType.DMA,
        pltpu.SemaphoreType.DMA,
    ],
)


CB = 2048


def _combine_body(d0_ref, s0_ref, d1_ref, s1_ref, out_ref):
    out_ref[...] = d0_ref[0] + d1_ref[0] - s0_ref[0] - s1_ref[0]


def _combine(fp):
    nb = NP // CB
    return pl.pallas_call(
        _combine_body,
        grid=(nb,),
        in_specs=[
            pl.BlockSpec((1, 3, CB), lambda i: (0, 0, i)),
            pl.BlockSpec((1, 3, CB), lambda i: (1, 0, i)),
            pl.BlockSpec((1, 3, CB), lambda i: (2, 0, i)),
            pl.BlockSpec((1, 3, CB), lambda i: (3, 0, i)),
        ],
        out_specs=pl.BlockSpec((3, CB), lambda i: (0, i)),
        out_shape=jax.ShapeDtypeStruct((3, NP), jnp.float32),
    )(fp, fp, fp, fp)


def kernel(pos, z, batch, edge_index, atom_embed, W_rbf, W_upd, W_e1, w_e2,
           W_f1, w_f2):
    src = edge_index[0].astype(jnp.int32)
    dst = edge_index[1].astype(jnp.int32)
    pad = jnp.zeros((EP - E,), jnp.int32)
    srcp = jnp.concatenate([src, pad])
    dstp = jnp.concatenate([dst, pad])
    dstc = jnp.concatenate([dst, jnp.full((EP - E,), N, jnp.int32)])
    px = jnp.asarray(pos[:, 0], jnp.float32)
    py = jnp.asarray(pos[:, 1], jnp.float32)
    pz = jnp.asarray(pos[:, 2], jnp.float32)
    zi = z.astype(jnp.int32)

    dist, zsrc, ux, uy, uz = _geom(srcp, dstp, zi, px, py, pz)

    apad = jnp.zeros((D, D), jnp.float32).at[:NZ].set(atom_embed)
    msg = _msg(dist.reshape(EP, 1), zsrc.astype(jnp.float32).reshape(EP, 1),
               W_rbf, apad)

    zeros128 = jnp.zeros((NP, D), jnp.float32)
    aggp = _scatter(msg, dstc, zeros128)

    zp = jnp.full((NP, 1), -1.0, jnp.float32).at[:N, 0].set(zi.astype(jnp.float32))
    bp = jnp.full((NP, 1), -1.0, jnp.float32).at[:N, 0].set(batch.astype(jnp.float32))
    energy1, fn = _node(aggp, zp, bp, apad, W_upd, W_e1, w_e2, W_f1, w_f2)

    bc = lax.bitcast_convert_type
    pk8 = jnp.stack([srcp, dstp, bc(ux, jnp.int32), bc(uy, jnp.int32),
                     bc(uz, jnp.int32), pad_col := jnp.zeros((EP,), jnp.int32),
                     pad_col, pad_col], axis=1)

    zerosn = jnp.zeros((NP,), jnp.float32)
    fp = _force(pk8, fn.reshape(NP)[:N], zerosn)

    fsum = _combine(fp.reshape(4, 3, NP))
    return (energy1.reshape(G), fsum.T[:N])


# speedup vs baseline: 1.2148x; 1.0602x over previous
import functools

import jax
import jax.numpy as jnp
from jax import lax
from jax.experimental import pallas as pl
from jax.experimental.pallas import tpu as pltpu
from jax.experimental.pallas import tpu_sc as plsc

N = 10000
E = 320000
D = 128
NRBF = 32
G = 256
NZ = 90
CUTOFF = 6.0

NC = 2
NS = 16
NW = NC * NS
EP = 327680
EW = EP // NW
K = 128
NCHUNK = EW // K
KE = 128
NCHUNKE = EW // KE
NP = 10240
NROWS = NP // NS


def _rsqrt16(x):
    i = lax.bitcast_convert_type(x, jnp.int32)
    i = jnp.int32(0x5F3759DF) - lax.shift_right_arithmetic(i, 1)
    y = lax.bitcast_convert_type(i, jnp.float32)
    for _ in range(3):
        y = y * (1.5 - 0.5 * x * y * y)
    return y


def _geom_body(src_hbm, dst_hbm, z_hbm, px_hbm, py_hbm, pz_hbm,
               dist_hbm, zsrc_hbm, ux_hbm, uy_hbm, uz_hbm,
               px_v, py_v, pz_v, z_v, src_v, dst_v, dist_v, zs_v,
               ux_v, uy_v, uz_v):
    cid = lax.axis_index("c")
    sid = lax.axis_index("s")
    wid = sid * NC + cid
    base = wid * EW

    pltpu.sync_copy(px_hbm, px_v)
    pltpu.sync_copy(py_hbm, py_v)
    pltpu.sync_copy(pz_hbm, pz_v)
    pltpu.sync_copy(z_hbm, z_v)
    pltpu.sync_copy(src_hbm.at[pl.ds(base, EW)], src_v)
    pltpu.sync_copy(dst_hbm.at[pl.ds(base, EW)], dst_v)

    def body(g, carry):
        off = g * 16
        s16 = src_v[pl.ds(off, 16)]
        d16 = dst_v[pl.ds(off, 16)]
        dx = plsc.load_gather(px_v, [s16]) - plsc.load_gather(px_v, [d16])
        dy = plsc.load_gather(py_v, [s16]) - plsc.load_gather(py_v, [d16])
        dz = plsc.load_gather(pz_v, [s16]) - plsc.load_gather(pz_v, [d16])
        zx = plsc.load_gather(z_v, [s16])
        sq = dx * dx + dy * dy + dz * dz + 1e-8
        r = _rsqrt16(sq)
        dist_v[pl.ds(off, 16)] = sq * r
        zs_v[pl.ds(off, 16)] = zx
        ux_v[pl.ds(off, 16)] = dx * r
        uy_v[pl.ds(off, 16)] = dy * r
        uz_v[pl.ds(off, 16)] = dz * r
        return carry

    lax.fori_loop(0, EW // 16, body, 0)
    pltpu.sync_copy(dist_v, dist_hbm.at[pl.ds(base, EW)])
    pltpu.sync_copy(zs_v, zsrc_hbm.at[pl.ds(base, EW)])
    pltpu.sync_copy(ux_v, ux_hbm.at[pl.ds(base, EW)])
    pltpu.sync_copy(uy_v, uy_hbm.at[pl.ds(base, EW)])
    pltpu.sync_copy(uz_v, uz_hbm.at[pl.ds(base, EW)])


_geom = pl.kernel(
    _geom_body,
    out_type=(jax.ShapeDtypeStruct((EP,), jnp.float32),
              jax.ShapeDtypeStruct((EP,), jnp.int32),
              jax.ShapeDtypeStruct((EP,), jnp.float32),
              jax.ShapeDtypeStruct((EP,), jnp.float32),
              jax.ShapeDtypeStruct((EP,), jnp.float32)),
    mesh=plsc.VectorSubcoreMesh(core_axis_name="c", subcore_axis_name="s"),
    compiler_params=pltpu.CompilerParams(needs_layout_passes=False),
    scratch_types=[
        pltpu.VMEM((N,), jnp.float32),
        pltpu.VMEM((N,), jnp.float32),
        pltpu.VMEM((N,), jnp.float32),
        pltpu.VMEM((N,), jnp.int32),
        pltpu.VMEM((EW,), jnp.int32),
        pltpu.VMEM((EW,), jnp.int32),
        pltpu.VMEM((EW,), jnp.float32),
        pltpu.VMEM((EW,), jnp.int32),
        pltpu.VMEM((EW,), jnp.float32),
        pltpu.VMEM((EW,), jnp.float32),
        pltpu.VMEM((EW,), jnp.float32),
    ],
)


BB = 5120


def _msg_body(dist_ref, zs_ref, wrbf_ref, apad_ref, out_ref):
    d = dist_ref[...]
    zs = zs_ref[...]
    centers = lax.broadcasted_iota(jnp.int32, (1, NRBF), 1).astype(jnp.float32) * (
        CUTOFF / (NRBF - 1))
    rbf = jnp.exp(-10.0 * (d - centers) ** 2)
    filt = jnp.dot(rbf, wrbf_ref[...], preferred_element_type=jnp.float32)
    zlane = lax.broadcasted_iota(jnp.int32, (1, D), 1).astype(jnp.float32)
    onehot = jnp.where(zs == zlane, 1.0, 0.0)
    h0 = jnp.dot(onehot, apad_ref[...], preferred_element_type=jnp.float32)
    x = h0 * filt
    out_ref[...] = x * jax.nn.sigmoid(x)


def _msg(distc, zsf, wrbf, apad):
    return pl.pallas_call(
        _msg_body,
        grid=(EP // BB,),
        in_specs=[
            pl.BlockSpec((BB, 1), lambda i: (i, 0)),
            pl.BlockSpec((BB, 1), lambda i: (i, 0)),
            pl.BlockSpec((NRBF, D), lambda i: (0, 0)),
            pl.BlockSpec((D, D), lambda i: (0, 0)),
        ],
        out_specs=pl.BlockSpec((BB, D), lambda i: (i, 0)),
        out_shape=jax.ShapeDtypeStruct((EP, D), jnp.float32),
    )(distc, zsf, wrbf, apad)


def _scatter_body(msg_hbm, dst_hbm, zeros_hbm, aggp_hbm,
                  dstk0_v, dstk1_v, msg0_v, msg1_v, agg_sh, sem0, sem1):
    cid = lax.axis_index("c")
    sid = lax.axis_index("s")
    pltpu.sync_copy(zeros_hbm.at[pl.ds(sid * NROWS, NROWS)],
                    agg_sh.at[pl.ds(sid * NROWS, NROWS)])
    plsc.subcore_barrier()

    ebase = cid * (EP // NC) + sid * EW
    bufs = ((dstk0_v, msg0_v, sem0), (dstk1_v, msg1_v, sem1))

    def start(c, b):
        dstk_v, msg_v, sem = bufs[b]
        pltpu.async_copy(dst_hbm.at[pl.ds(ebase + c * K, K)], dstk_v, sem)
        pltpu.async_copy(msg_hbm.at[pl.ds(ebase + c * K, K)], msg_v, sem)

    def finish(c, b, prefetch):
        dstk_v, msg_v, sem = bufs[b]
        pltpu.make_async_copy(dst_hbm.at[pl.ds(ebase, K)], dstk_v, sem).wait()
        pltpu.make_async_copy(msg_hbm.at[pl.ds(ebase, K)], msg_v, sem).wait()
        pltpu.sync_copy(msg_v, agg_sh.at[dstk_v], add=True)
        if prefetch:
            @pl.when(c + 2 < NCHUNK)
            def _():
                start(c + 2, b)

    start(0, 0)
    start(1, 1)

    def body(i, carry):
        finish(2 * i, 0, True)
        finish(2 * i + 1, 1, True)
        return carry

    lax.fori_loop(0, NCHUNK // 2, body, 0)
    plsc.subcore_barrier()
    pltpu.sync_copy(agg_sh.at[pl.ds(sid * NROWS, NROWS)],
                    aggp_hbm.at[pl.ds(cid * NP + sid * NROWS, NROWS)])


_scatter = pl.kernel(
    _scatter_body,
    out_type=jax.ShapeDtypeStruct((NC * NP, D), jnp.float32),
    mesh=plsc.VectorSubcoreMesh(core_axis_name="c", subcore_axis_name="s"),
    compiler_params=pltpu.CompilerParams(needs_layout_passes=False),
    scratch_types=[
        pltpu.VMEM((K,), jnp.int32),
        pltpu.VMEM((K,), jnp.int32),
        pltpu.VMEM((K, D), jnp.float32),
        pltpu.VMEM((K, D), jnp.float32),
        pltpu.VMEM_SHARED((NP, D), jnp.float32),
        pltpu.SemaphoreType.DMA,
        pltpu.SemaphoreType.DMA,
    ],
)


BN = 2048


def _node_body(agg0_ref, agg1_ref, z_ref, b_ref, apad_ref, wupd_ref,
               we1_ref, we2_ref, wf1_ref, wf2_ref, e_ref, fn_ref):
    i = pl.program_id(0)
    zf = z_ref[...]
    zlane = lax.broadcasted_iota(jnp.int32, (1, D), 1).astype(jnp.float32)
    onehot = jnp.where(zf == zlane, 1.0, 0.0)
    h0 = jnp.dot(onehot, apad_ref[...], preferred_element_type=jnp.float32)
    agg = agg0_ref[...] + agg1_ref[...]
    u = jnp.dot(agg, wupd_ref[...], preferred_element_type=jnp.float32)
    h = h0 + u * jax.nn.sigmoid(u)
    e1 = jnp.dot(h, we1_ref[...], preferred_element_type=jnp.float32)
    e1 = e1 * jax.nn.sigmoid(e1)
    ne = jnp.dot(e1, we2_ref[...], preferred_element_type=jnp.float32)
    f1 = jnp.dot(h, wf1_ref[...], preferred_element_type=jnp.float32)
    f1 = f1 * jax.nn.sigmoid(f1)
    fn = jnp.dot(f1, wf2_ref[...], preferred_element_type=jnp.float32)
    fn_ref[...] = fn
    glane = lax.broadcasted_iota(jnp.int32, (1, G), 1).astype(jnp.float32)
    contrib = jnp.where(b_ref[...] == glane, ne, 0.0)
    part = jnp.sum(contrib, axis=0, keepdims=True)

    @pl.when(i == 0)
    def _():
        e_ref[...] = jnp.zeros_like(e_ref)

    e_ref[...] += part


def _node(aggp, zf, bf, apad, wupd, we1, we2, wf1, wf2):
    nb = NP // BN
    return pl.pallas_call(
        _node_body,
        grid=(nb,),
        in_specs=[
            pl.BlockSpec((BN, D), lambda i: (i, 0)),
            pl.BlockSpec((BN, D), lambda i, nb=nb: (i + nb, 0)),
            pl.BlockSpec((BN, 1), lambda i: (i, 0)),
            pl.BlockSpec((BN, 1), lambda i: (i, 0)),
            pl.BlockSpec((D, D), lambda i: (0, 0)),
            pl.BlockSpec((D, D), lambda i: (0, 0)),
            pl.BlockSpec((D, D), lambda i: (0, 0)),
            pl.BlockSpec((D, 1), lambda i: (0, 0)),
            pl.BlockSpec((D, D), lambda i: (0, 0)),
            pl.BlockSpec((D, 1), lambda i: (0, 0)),
        ],
        out_specs=[
            pl.BlockSpec((1, G), lambda i: (0, 0)),
            pl.BlockSpec((BN, 1), lambda i: (i, 0)),
        ],
        out_shape=[
            jax.ShapeDtypeStruct((1, G), jnp.float32),
            jax.ShapeDtypeStruct((NP, 1), jnp.float32),
        ],
    )(aggp, aggp, zf, bf, apad, wupd, we1, we2, wf1, wf2)


def _force_body(pk_hbm, fn_hbm, zeros_hbm, fp_hbm,
                fn_v, pk0_v, pk1_v, sk0_v, sk1_v, dk0_v, dk1_v,
                fx0_v, fy0_v, fz0_v, fx1_v, fy1_v, fz1_v,
                fxd_sh, fyd_sh, fzd_sh, fxs_sh, fys_sh, fzs_sh,
                semi0, semi1, sems0, sems1):
    cid = lax.axis_index("c")
    sid = lax.axis_index("s")
    rs = sid * NROWS
    for tab in (fxd_sh, fyd_sh, fzd_sh, fxs_sh, fys_sh, fzs_sh):
        pltpu.sync_copy(zeros_hbm.at[pl.ds(rs, NROWS)], tab.at[pl.ds(rs, NROWS)])
    pltpu.sync_copy(fn_hbm, fn_v)
    plsc.subcore_barrier()

    ebase = cid * (EP // NC) + sid * EW
    lane = lax.iota(jnp.int32, 16)
    czero = jnp.zeros((16,), jnp.int32)
    bufs = ((pk0_v, sk0_v, dk0_v, fx0_v, fy0_v, fz0_v, semi0, sems0),
            (pk1_v, sk1_v, dk1_v, fx1_v, fy1_v, fz1_v, semi1, sems1))

    def start_in(c, b):
        pk_v, _, _, _, _, _, semi, _ = bufs[b]
        pltpu.async_copy(pk_hbm.at[pl.ds(ebase + c * KE, KE)], pk_v, semi)

    def scat_pairs(b):
        _, sk_v, dk_v, fx_v, fy_v, fz_v, _, _ = bufs[b]
        return ((fx_v, fxd_sh, dk_v), (fy_v, fyd_sh, dk_v), (fz_v, fzd_sh, dk_v),
                (fx_v, fxs_sh, sk_v), (fy_v, fys_sh, sk_v), (fz_v, fzs_sh, sk_v))

    def process(c, b, first, prefetch):
        pk_v, sk_v, dk_v, fx_v, fy_v, fz_v, semi, sems = bufs[b]
        pltpu.make_async_copy(pk_hbm.at[pl.ds(ebase, KE)], pk_v, semi).wait()
        del first
        for g in range(KE // 16):
            off = g * 16
            rows = off + lane
            s16 = plsc.load_gather(pk_v, [rows, czero])
            d16 = plsc.load_gather(pk_v, [rows, czero + 1])
            ux = plsc.bitcast(plsc.load_gather(pk_v, [rows, czero + 2]), jnp.float32)
            uy = plsc.bitcast(plsc.load_gather(pk_v, [rows, czero + 3]), jnp.float32)
            uz = plsc.bitcast(plsc.load_gather(pk_v, [rows, czero + 4]), jnp.float32)
            fnd = plsc.load_gather(fn_v, [d16])
            sk_v[pl.ds(off, 16)] = s16
            dk_v[pl.ds(off, 16)] = d16
            fx_v[pl.ds(off, 16)] = fnd * ux
            fy_v[pl.ds(off, 16)] = fnd * uy
            fz_v[pl.ds(off, 16)] = fnd * uz
        for val, tab, idx in scat_pairs(b):
            pltpu.sync_copy(val, tab.at[idx], add=True)
        if prefetch:
            @pl.when(c + 2 < NCHUNKE)
            def _():
                start_in(c + 2, b)

    start_in(0, 0)
    start_in(1, 1)
    process(0, 0, True, True)
    process(1, 1, True, True)

    def body(i, carry):
        process(2 * i + 2, 0, False, True)
        process(2 * i + 3, 1, False, True)
        return carry

    lax.fori_loop(0, (NCHUNKE - 2) // 2, body, 0)
    plsc.subcore_barrier()
    fbase = cid * 6 * NP
    for t, tab in enumerate((fxd_sh, fyd_sh, fzd_sh, fxs_sh, fys_sh, fzs_sh)):
        pltpu.sync_copy(tab.at[pl.ds(rs, NROWS)],
                        fp_hbm.at[pl.ds(fbase + t * NP + rs, NROWS)])


_force = pl.kernel(
    _force_body,
    out_type=jax.ShapeDtypeStruct((NC * 6 * NP,), jnp.float32),
    mesh=plsc.VectorSubcoreMesh(core_axis_name="c", subcore_axis_name="s"),
    compiler_params=pltpu.CompilerParams(needs_layout_passes=False),
    scratch_types=[
        pltpu.VMEM((N,), jnp.float32),
        pltpu.VMEM((KE, 8), jnp.int32),
        pltpu.VMEM((KE, 8), jnp.int32),
        pltpu.VMEM((KE,), jnp.int32),
        pltpu.VMEM((KE,), jnp.int32),
        pltpu.VMEM((KE,), jnp.int32),
        pltpu.VMEM((KE,), jnp.int32),
        pltpu.VMEM((KE,), jnp.float32),
        pltpu.VMEM((KE,), jnp.float32),
        pltpu.VMEM((KE,), jnp.float32),
        pltpu.VMEM((KE,), jnp.float32),
        pltpu.VMEM((KE,), jnp.float32),
        pltpu.VMEM((KE,), jnp.float32),
        pltpu.VMEM_SHARED((NP,), jnp.float32),
        pltpu.VMEM_SHARED((NP,), jnp.float32),
        pltpu.VMEM_SHARED((NP,), jnp.float32),
        pltpu.VMEM_SHARED((NP,), jnp.float32),
        pltpu.VMEM_SHARED((NP,), jnp.float32),
        pltpu.VMEM_SHARED((NP,), jnp.float32),
        pltpu.SemaphoreType.DMA,
        pltpu.SemaphoreType.DMA,
        pltpu.SemaphoreType.DMA,
        pltpu.SemaphoreType.DMA,
    ],
)


CB = 2048


def _combine_body(d0_ref, s0_ref, d1_ref, s1_ref, out_ref):
    out_ref[...] = d0_ref[0] + d1_ref[0] - s0_ref[0] - s1_ref[0]


def _combine(fp):
    nb = NP // CB
    return pl.pallas_call(
        _combine_body,
        grid=(nb,),
        in_specs=[
            pl.BlockSpec((1, 3, CB), lambda i: (0, 0, i)),
            pl.BlockSpec((1, 3, CB), lambda i: (1, 0, i)),
            pl.BlockSpec((1, 3, CB), lambda i: (2, 0, i)),
            pl.BlockSpec((1, 3, CB), lambda i: (3, 0, i)),
        ],
        out_specs=pl.BlockSpec((3, CB), lambda i: (0, i)),
        out_shape=jax.ShapeDtypeStruct((3, NP), jnp.float32),
    )(fp, fp, fp, fp)


def kernel(pos, z, batch, edge_index, atom_embed, W_rbf, W_upd, W_e1, w_e2,
           W_f1, w_f2):
    src = edge_index[0].astype(jnp.int32)
    dst = edge_index[1].astype(jnp.int32)
    pad = jnp.zeros((EP - E,), jnp.int32)
    srcp = jnp.concatenate([src, pad])
    dstp = jnp.concatenate([dst, pad])
    dstc = jnp.concatenate([dst, jnp.full((EP - E,), N, jnp.int32)])
    px = jnp.asarray(pos[:, 0], jnp.float32)
    py = jnp.asarray(pos[:, 1], jnp.float32)
    pz = jnp.asarray(pos[:, 2], jnp.float32)
    zi = z.astype(jnp.int32)

    dist, zsrc, ux, uy, uz = _geom(srcp, dstp, zi, px, py, pz)

    apad = jnp.zeros((D, D), jnp.float32).at[:NZ].set(atom_embed)
    msg = _msg(dist.reshape(EP, 1), zsrc.astype(jnp.float32).reshape(EP, 1),
               W_rbf, apad)

    zeros128 = jnp.zeros((NP, D), jnp.float32)
    aggp = _scatter(msg, dstc, zeros128)

    zp = jnp.full((NP, 1), -1.0, jnp.float32).at[:N, 0].set(zi.astype(jnp.float32))
    bp = jnp.full((NP, 1), -1.0, jnp.float32).at[:N, 0].set(batch.astype(jnp.float32))
    energy1, fn = _node(aggp, zp, bp, apad, W_upd, W_e1, w_e2, W_f1, w_f2)

    bc = lax.bitcast_convert_type
    pk8 = jnp.stack([srcp, dstp, bc(ux, jnp.int32), bc(uy, jnp.int32),
                     bc(uz, jnp.int32), pad_col := jnp.zeros((EP,), jnp.int32),
                     pad_col, pad_col], axis=1)

    zerosn = jnp.zeros((NP,), jnp.float32)
    fp = _force(pk8, fn.reshape(NP)[:N], zerosn)

    fsum = _combine(fp.reshape(4, 3, NP))
    return (energy1.reshape(G), fsum.T[:N])
